# baseline (speedup 1.0000x reference)
import functools
import jax
import jax.numpy as jnp
from jax import lax
from jax.experimental import pallas as pl
from jax.experimental.pallas import tpu as pltpu
from jax.experimental.pallas import tpu_sc as plsc

BATCH = 4
SEQ = 2048
D_MODEL = 1024
L = 16
N_W = 32
ROWS_PER_W = BATCH * SEQ // N_W
CH = 16
N_CH = ROWS_PER_W // CH
UNROLL = 8


def kernel(x, pos_table):
    xf = x.reshape(BATCH * SEQ, D_MODEL)
    mesh = plsc.VectorSubcoreMesh(core_axis_name="core", subcore_axis_name="subcore")

    @functools.partial(
        pl.kernel,
        out_type=jax.ShapeDtypeStruct((BATCH * SEQ, D_MODEL), jnp.float32),
        mesh=mesh,
        scratch_types=[
            pltpu.VMEM((CH, D_MODEL), jnp.float32),
            pltpu.VMEM((CH, D_MODEL), jnp.float32),
            pltpu.VMEM((CH, D_MODEL), jnp.float32),
            pltpu.VMEM((CH, D_MODEL), jnp.float32),
            pltpu.VMEM((CH, D_MODEL), jnp.float32),
            pltpu.VMEM((CH, D_MODEL), jnp.float32),
            pltpu.SemaphoreType.DMA,
            pltpu.SemaphoreType.DMA,
            pltpu.SemaphoreType.DMA,
            pltpu.SemaphoreType.DMA,
            pltpu.SemaphoreType.DMA,
            pltpu.SemaphoreType.DMA,
        ],
    )
    def k(x_hbm, pos_hbm, o_hbm,
          xb0, xb1, pb0, pb1, ob0, ob1,
          sx0, sx1, sp0, sp1, so0, so1):
        wid = lax.axis_index("subcore") * 2 + lax.axis_index("core")
        base = wid * ROWS_PER_W
        pos_base = lax.rem(base, SEQ)

        xb = (xb0, xb1)
        pb = (pb0, pb1)
        ob = (ob0, ob1)
        sx = (sx0, sx1)
        sp = (sp0, sp1)
        so = (so0, so1)

        def start_in(g):
            p = g & 1
            r = base + g * CH
            q = pos_base + g * CH
            hx = pltpu.async_copy(x_hbm.at[pl.ds(r, CH), :], xb[p], sx[p])
            hp = pltpu.async_copy(pos_hbm.at[pl.ds(q, CH), :], pb[p], sp[p])
            return hx, hp

        def compute(p):
            @pl.loop(0, CH)
            def _(row):
                @pl.loop(0, D_MODEL, step=L * UNROLL)
                def _(c):
                    for u in range(UNROLL):
                        slc = (pl.ds(row, 1), pl.ds(c + u * L, L))
                        ob[p].at[*slc][...] = (
                            xb[p].at[*slc][...] + pb[p].at[*slc][...])

        in_flight = {0: start_in(0)}
        out_flight = {}
        for g in range(N_CH):
            p = g & 1
            if g + 1 < N_CH:
                in_flight[g + 1] = start_in(g + 1)
            hx, hp = in_flight.pop(g)
            hx.wait()
            hp.wait()
            if g >= 2:
                out_flight.pop(g - 2).wait()
            compute(p)
            r = base + g * CH
            out_flight[g] = pltpu.async_copy(ob[p], o_hbm.at[pl.ds(r, CH), :], so[p])
        out_flight.pop(N_CH - 2).wait()
        out_flight.pop(N_CH - 1).wait()

    return k(xf, pos_table).reshape(BATCH, SEQ, D_MODEL)


# speedup vs baseline: 4.8701x; 4.8701x over previous
import jax
import jax.numpy as jnp
from jax.experimental import pallas as pl

BATCH = 4
SEQ = 2048
D_MODEL = 1024
RB = 1024


def _add_body(x_ref, pos_ref, o_ref):
    i = pl.program_id(0)
    n_per_batch = SEQ // RB
    s0 = (i % n_per_batch) * RB
    o_ref[...] = x_ref[...] + pos_ref[pl.ds(s0, RB), :]


def kernel(x, pos_table):
    n_rows = BATCH * SEQ
    out = pl.pallas_call(
        _add_body,
        grid=(n_rows // RB,),
        in_specs=[
            pl.BlockSpec((RB, D_MODEL), lambda i: (i, 0)),
            pl.BlockSpec((SEQ, D_MODEL), lambda i: (0, 0)),
        ],
        out_specs=pl.BlockSpec((RB, D_MODEL), lambda i: (i, 0)),
        out_shape=jax.ShapeDtypeStruct((n_rows, D_MODEL), jnp.float32),
    )(x.reshape(n_rows, D_MODEL), pos_table)
    return out.reshape(BATCH, SEQ, D_MODEL)
